# baseline (speedup 1.0000x reference)
import functools

import jax
import jax.numpy as jnp
from jax import lax
from jax.experimental import pallas as pl
from jax.experimental.pallas import tpu as pltpu
from jax.experimental.pallas import tpu_sc as plsc

N_ENT = 100000
N_REL = 1000
DIM = 128
B = 16384
BETA = 1.0
EPS = 1e-23
REG_DELTA = 0.05
REG_MIN = 0.0005
REG_REL = 0.0005

_NC = 2
_NS = 16
_NW = _NC * _NS
_CHUNK = 128
_NSPLIT = 2


def _sc_gather_body(nrows, split, refs):
    (groups, idx_all, bufs, isem, gsems, ssems) = refs
    _NCHUNK = nrows // _NW // _CHUNK
    wid = lax.axis_index("s") * _NC + lax.axis_index("c")
    base = wid * (nrows // _NW)
    gbase = split * nrows + base

    idx_copies = {}
    for g, (idx_hbm, _) in enumerate(groups):
        for c in range(_NCHUNK):
            j = g * _NCHUNK + c
            idx_copies[j] = pltpu.async_copy(
                idx_hbm.at[pl.ds(gbase + c * _CHUNK, _CHUNK)],
                idx_all.at[j], isem)

    units = []
    for c in range(_NCHUNK):
        for g, (_, pairs) in enumerate(groups):
            for table, out in pairs:
                units.append((g * _NCHUNK + c, table, out, base + c * _CHUNK))

    nbuf = len(gsems)
    gathers = [None] * nbuf
    stores = [None] * nbuf
    idx_done = set()

    def start_gather(k):
        slot = k % nbuf
        j, table, _, _ = units[k]
        if stores[slot] is not None:
            stores[slot].wait()
        if j not in idx_done:
            idx_copies[j].wait()
            idx_done.add(j)
        gathers[slot] = pltpu.async_copy(
            table.at[idx_all.at[j]], bufs.at[slot], gsems[slot])

    start_gather(0)
    for k in range(len(units)):
        if k + 1 < len(units):
            start_gather(k + 1)
        slot = k % nbuf
        _, _, out, row0 = units[k]
        gathers[slot].wait()
        stores[slot] = pltpu.async_copy(
            bufs.at[slot], out.at[pl.ds(row0, _CHUNK)], ssems[slot])
    for st in stores:
        if st is not None:
            st.wait()


_NBUF = 6
_REL_ON_SC = 4


def _make_sc_gather(nrows, split):
    row = jax.ShapeDtypeStruct((nrows, DIM), jnp.float32)
    nchunk = nrows // _NW // _CHUNK
    n_out = 8 + _REL_ON_SC

    def body(*refs):
        n_tab = 2 + _REL_ON_SC
        n_idx = 4 + (1 if _REL_ON_SC else 0)
        tabs = refs[:n_tab]
        idxs = refs[n_tab:n_tab + n_idx]
        outs = refs[n_tab + n_idx:n_tab + n_idx + n_out]
        idx_all, bufs, isem, gsems, ssems = refs[n_tab + n_idx + n_out:]
        min_e, delta_e = tabs[0], tabs[1]
        groups = [
            (idxs[0], ((min_e, outs[0]), (delta_e, outs[1]))),
            (idxs[1], ((min_e, outs[2]), (delta_e, outs[3]))),
            (idxs[2], ((min_e, outs[4]), (delta_e, outs[5]))),
            (idxs[3], ((min_e, outs[6]), (delta_e, outs[7]))),
        ]
        if _REL_ON_SC:
            groups.append((idxs[4], tuple(
                (tabs[2 + k], outs[8 + k]) for k in range(_REL_ON_SC))))
        _sc_gather_body(nrows, split,
                        (groups, idx_all, bufs, isem, gsems, ssems))

    return pl.kernel(
        body,
        out_type=[row] * n_out,
        mesh=plsc.VectorSubcoreMesh(core_axis_name="c", subcore_axis_name="s"),
        scratch_types=[
            pltpu.VMEM((5 * nchunk, _CHUNK), jnp.int32),
            pltpu.VMEM((_NBUF, _CHUNK, DIM), jnp.float32),
            pltpu.SemaphoreType.DMA,
            [pltpu.SemaphoreType.DMA] * _NBUF,
            [pltpu.SemaphoreType.DMA] * _NBUF,
        ],
    )


def _log1p(x):
    return jnp.where(x < 1e-6, x, jnp.log(1.0 + x))


def _logaddexp(a, b):
    mx = jnp.maximum(a, b)
    return mx + _log1p(jnp.exp(-jnp.abs(a - b)))


def _vol_logs(bmin, bmax):
    d = (bmax - bmin) / BETA
    sp = jnp.maximum(d, 0.0) + _log1p(jnp.exp(-jnp.abs(d)))
    t = sp * BETA + EPS
    half = t.shape[1] // 2
    v = t[:, :half] * t[:, half:]
    quarter = half // 2
    v = v[:, :quarter] * v[:, quarter:]
    return jnp.log(v)


_BB = 512


_NRELP = 1024


def _tc_loss_body(nb, *args):
    k = _REL_ON_SC
    (mh, dh, mt, dt, mnh, dnh, mnt, dnt) = args[:8]
    screl = args[8:8 + k]
    rest = args[8 + k:]
    if k < 4:
        rel_hi, rel_lo, rv, conf, out_ref, acc_ref = rest
    else:
        conf, out_ref, acc_ref = rest
    i = pl.program_id(0)

    @pl.when(i == 0)
    def _():
        acc_ref[0] = 0.0

    relrows = [r[...] for r in screl]
    if k < 4:
        cols = jax.lax.broadcasted_iota(jnp.int32, (_BB, _NRELP), 1)
        onehot = (cols == rv[...]).astype(jnp.bfloat16)
        rows = (jnp.dot(onehot, rel_hi[...], preferred_element_type=jnp.float32)
                + jnp.dot(onehot, rel_lo[...], preferred_element_type=jnp.float32))
        for j in range(4 - k):
            relrows.append(rows[:, j * DIM:(j + 1) * DIM])
    rth, rsh, rtt, rst = relrows

    sc_h = jnp.exp(rsh)
    sc_t = jnp.exp(rst)
    edh = jnp.exp(dh[...])
    edt = jnp.exp(dt[...])

    def meet(a_min, a_max, b_min, b_max):
        mmin = BETA * _logaddexp(a_min / BETA, b_min / BETA)
        mmax = -BETA * _logaddexp(-a_max / BETA, -b_max / BETA)
        return mmin, mmax

    h_min = mh[...] * sc_h + rth
    h_max = h_min + edh * sc_h
    t_min = mt[...] * sc_t + rtt
    t_max = t_min + edt * sc_t
    pmin, pmax = meet(h_min, h_max, t_min, t_max)

    nh_min = mnh[...] * sc_h + rth
    nh_max = nh_min + jnp.exp(dnh[...]) * sc_h
    nt_min = mnt[...] * sc_t + rtt
    nt_max = nt_min + jnp.exp(dnt[...]) * sc_t
    qmin, qmax = meet(nh_min, nh_max, nt_min, nt_max)

    L = jnp.concatenate(
        [_vol_logs(pmin, pmax), _vol_logs(t_min, t_max),
         _vol_logs(qmin, qmax), _vol_logs(nt_min, nt_max)], axis=1)
    qdim = DIM // 4
    g2 = jax.lax.broadcasted_iota(jnp.int32, (4 * qdim, 2), 0) // qdim
    c2 = jax.lax.broadcasted_iota(jnp.int32, (4 * qdim, 2), 1)
    w2 = (jnp.where(g2 == 2 * c2, 1.0, 0.0)
          - jnp.where(g2 == 2 * c2 + 1, 1.0, 0.0)).astype(jnp.float32)
    lr = jnp.dot(L, w2, preferred_element_type=jnp.float32)
    p = jnp.exp(jnp.minimum(lr, 0.0))
    x = p[:, 0:1] - conf[...]
    se_row = x * x + p[:, 1:2] * p[:, 1:2]

    sqs = [edh, edt, mh[...], mt[...], jnp.exp(rth), jnp.exp(rtt),
           sc_h, sc_t]
    Y = jnp.concatenate([a * a for a in sqs], axis=1)
    g8 = jax.lax.broadcasted_iota(jnp.int32, (8 * DIM, 8), 0) // DIM
    c8 = jax.lax.broadcasted_iota(jnp.int32, (8 * DIM, 8), 1)
    w8 = (g8 == c8).astype(jnp.float32)
    S = jnp.dot(Y, w8, preferred_element_type=jnp.float32)
    norms = jnp.sqrt(S)
    wreg = jnp.where(
        jax.lax.broadcasted_iota(jnp.int32, norms.shape, 1) < 2,
        REG_DELTA, REG_MIN)
    acc_ref[0] += jnp.sum(se_row) + jnp.sum(norms * wreg)

    @pl.when(i == nb - 1)
    def _():
        out_ref[...] = jnp.full((1, 1), acc_ref[0], jnp.float32)


def _make_tc_loss(nrows, split):
    nb = nrows // _BB
    off = split * nb
    k = _REL_ON_SC
    row_spec = pl.BlockSpec((_BB, DIM), lambda i: (i, 0))
    rel_spec = pl.BlockSpec((_NRELP, (4 - k) * DIM), lambda i: (0, 0))
    col_spec = pl.BlockSpec((_BB, 1), lambda i: (i + off, 0))
    in_specs = [row_spec] * (8 + k)
    if k < 4:
        in_specs += [rel_spec] * 2 + [col_spec]
    in_specs += [col_spec]
    return pl.pallas_call(
        functools.partial(_tc_loss_body, nb),
        grid=(nb,),
        in_specs=in_specs,
        out_specs=pl.BlockSpec((1, 1), lambda i: (0, 0)),
        out_shape=jax.ShapeDtypeStruct((1, 1), jnp.float32),
        scratch_shapes=[pltpu.SMEM((1,), jnp.float32)],
    )


def kernel(ids, negative_samples, confidence, min_embedding, delta_embedding,
           rel_trans_for_head, rel_scale_for_head, rel_trans_for_tail,
           rel_scale_for_tail):
    ids = ids.astype(jnp.int32)
    neg = negative_samples.astype(jnp.int32)
    h = ids[:, 0]
    t = ids[:, 2]
    nh = neg[:, 0]
    nt = neg[:, 2]
    r = ids[:, 1]
    n = B // _NSPLIT
    k = _REL_ON_SC
    conf2d = confidence.reshape(B, 1)
    rel_tables = [rel_trans_for_head, rel_scale_for_head,
                  rel_trans_for_tail, rel_scale_for_tail]
    if k < 4:
        r2d = r.reshape(B, 1)
        rel_cat = jnp.concatenate(rel_tables[k:], axis=1)
        rel_cat = jnp.pad(rel_cat, ((0, _NRELP - N_REL), (0, 0)))
        rel_hi = rel_cat.astype(jnp.bfloat16)
        rel_lo = (rel_cat - rel_hi.astype(jnp.float32)).astype(jnp.bfloat16)
    gathered_chunks = []
    for s in range(_NSPLIT):
        sc_args = ([min_embedding, delta_embedding] + rel_tables[:k]
                   + [h, t, nh, nt] + ([r] if k else []))
        gathered_chunks.append(_make_sc_gather(n, s)(*sc_args))
    partials = []
    for s in range(_NSPLIT):
        tc_args = list(gathered_chunks[s])
        if k < 4:
            tc_args += [rel_hi, rel_lo, r2d]
        tc_args.append(conf2d)
        partials.append(_make_tc_loss(n, s)(*tc_args))
    total = partials[0]
    for p in partials[1:]:
        total = total + p
    return (total * (1.0 / B)).reshape(())


# speedup vs baseline: 1.0254x; 1.0254x over previous
import functools

import jax
import jax.numpy as jnp
from jax import lax
from jax.experimental import pallas as pl
from jax.experimental.pallas import tpu as pltpu
from jax.experimental.pallas import tpu_sc as plsc

N_ENT = 100000
N_REL = 1000
DIM = 128
B = 16384
BETA = 1.0
EPS = 1e-23
REG_DELTA = 0.05
REG_MIN = 0.0005
REG_REL = 0.0005

_NC = 2
_NS = 16
_NW = _NC * _NS
_CHUNK = 128
_NSPLIT = 4


def _sc_gather_body(nrows, split, refs):
    (groups, idx_all, bufs, isem, gsems, ssems) = refs
    _NCHUNK = nrows // _NW // _CHUNK
    wid = lax.axis_index("s") * _NC + lax.axis_index("c")
    base = wid * (nrows // _NW)
    gbase = split * nrows + base

    idx_copies = {}
    for g, (idx_hbm, _) in enumerate(groups):
        for c in range(_NCHUNK):
            j = g * _NCHUNK + c
            idx_copies[j] = pltpu.async_copy(
                idx_hbm.at[pl.ds(gbase + c * _CHUNK, _CHUNK)],
                idx_all.at[j], isem)

    units = []
    for c in range(_NCHUNK):
        for g, (_, pairs) in enumerate(groups):
            for table, out in pairs:
                units.append((g * _NCHUNK + c, table, out, base + c * _CHUNK))

    nbuf = len(gsems)
    gathers = [None] * nbuf
    stores = [None] * nbuf
    idx_done = set()

    def start_gather(k):
        slot = k % nbuf
        j, table, _, _ = units[k]
        if stores[slot] is not None:
            stores[slot].wait()
        if j not in idx_done:
            idx_copies[j].wait()
            idx_done.add(j)
        gathers[slot] = pltpu.async_copy(
            table.at[idx_all.at[j]], bufs.at[slot], gsems[slot])

    start_gather(0)
    for k in range(len(units)):
        if k + 1 < len(units):
            start_gather(k + 1)
        slot = k % nbuf
        _, _, out, row0 = units[k]
        gathers[slot].wait()
        stores[slot] = pltpu.async_copy(
            bufs.at[slot], out.at[pl.ds(row0, _CHUNK)], ssems[slot])
    for st in stores:
        if st is not None:
            st.wait()


_NBUF = 6
_REL_ON_SC = 4


def _make_sc_gather(nrows, split):
    row = jax.ShapeDtypeStruct((nrows, DIM), jnp.float32)
    nchunk = nrows // _NW // _CHUNK
    n_out = 8 + _REL_ON_SC

    def body(*refs):
        n_tab = 2 + _REL_ON_SC
        n_idx = 4 + (1 if _REL_ON_SC else 0)
        tabs = refs[:n_tab]
        idxs = refs[n_tab:n_tab + n_idx]
        outs = refs[n_tab + n_idx:n_tab + n_idx + n_out]
        idx_all, bufs, isem, gsems, ssems = refs[n_tab + n_idx + n_out:]
        min_e, delta_e = tabs[0], tabs[1]
        groups = [
            (idxs[0], ((min_e, outs[0]), (delta_e, outs[1]))),
            (idxs[1], ((min_e, outs[2]), (delta_e, outs[3]))),
            (idxs[2], ((min_e, outs[4]), (delta_e, outs[5]))),
            (idxs[3], ((min_e, outs[6]), (delta_e, outs[7]))),
        ]
        if _REL_ON_SC:
            groups.append((idxs[4], tuple(
                (tabs[2 + k], outs[8 + k]) for k in range(_REL_ON_SC))))
        _sc_gather_body(nrows, split,
                        (groups, idx_all, bufs, isem, gsems, ssems))

    return pl.kernel(
        body,
        out_type=[row] * n_out,
        mesh=plsc.VectorSubcoreMesh(core_axis_name="c", subcore_axis_name="s"),
        scratch_types=[
            pltpu.VMEM((5 * nchunk, _CHUNK), jnp.int32),
            pltpu.VMEM((_NBUF, _CHUNK, DIM), jnp.float32),
            pltpu.SemaphoreType.DMA,
            [pltpu.SemaphoreType.DMA] * _NBUF,
            [pltpu.SemaphoreType.DMA] * _NBUF,
        ],
    )


def _log1p(x):
    return jnp.where(x < 1e-6, x, jnp.log(1.0 + x))


def _logaddexp(a, b):
    mx = jnp.maximum(a, b)
    return mx + _log1p(jnp.exp(-jnp.abs(a - b)))


def _vol_logs(bmin, bmax):
    d = (bmax - bmin) / BETA
    sp = jnp.maximum(d, 0.0) + _log1p(jnp.exp(-jnp.abs(d)))
    t = sp * BETA + EPS
    half = t.shape[1] // 2
    v = t[:, :half] * t[:, half:]
    quarter = half // 2
    v = v[:, :quarter] * v[:, quarter:]
    return jnp.log(v)


_BB = 1024


_NRELP = 1024


def _tc_loss_body(nb, *args):
    k = _REL_ON_SC
    (mh, dh, mt, dt, mnh, dnh, mnt, dnt) = args[:8]
    screl = args[8:8 + k]
    rest = args[8 + k:]
    if k < 4:
        rel_hi, rel_lo, rv, conf, out_ref, acc_ref = rest
    else:
        conf, out_ref, acc_ref = rest
    i = pl.program_id(0)

    @pl.when(i == 0)
    def _():
        acc_ref[0] = 0.0

    relrows = [r[...] for r in screl]
    if k < 4:
        cols = jax.lax.broadcasted_iota(jnp.int32, (_BB, _NRELP), 1)
        onehot = (cols == rv[...]).astype(jnp.bfloat16)
        rows = (jnp.dot(onehot, rel_hi[...], preferred_element_type=jnp.float32)
                + jnp.dot(onehot, rel_lo[...], preferred_element_type=jnp.float32))
        for j in range(4 - k):
            relrows.append(rows[:, j * DIM:(j + 1) * DIM])
    rth, rsh, rtt, rst = relrows

    sc_h = jnp.exp(rsh)
    sc_t = jnp.exp(rst)
    edh = jnp.exp(dh[...])
    edt = jnp.exp(dt[...])

    def meet(a_min, a_max, b_min, b_max):
        mmin = BETA * _logaddexp(a_min / BETA, b_min / BETA)
        mmax = -BETA * _logaddexp(-a_max / BETA, -b_max / BETA)
        return mmin, mmax

    h_min = mh[...] * sc_h + rth
    h_max = h_min + edh * sc_h
    t_min = mt[...] * sc_t + rtt
    t_max = t_min + edt * sc_t
    pmin, pmax = meet(h_min, h_max, t_min, t_max)

    nh_min = mnh[...] * sc_h + rth
    nh_max = nh_min + jnp.exp(dnh[...]) * sc_h
    nt_min = mnt[...] * sc_t + rtt
    nt_max = nt_min + jnp.exp(dnt[...]) * sc_t
    qmin, qmax = meet(nh_min, nh_max, nt_min, nt_max)

    L = jnp.concatenate(
        [_vol_logs(pmin, pmax), _vol_logs(t_min, t_max),
         _vol_logs(qmin, qmax), _vol_logs(nt_min, nt_max)], axis=1)
    qdim = DIM // 4
    g2 = jax.lax.broadcasted_iota(jnp.int32, (4 * qdim, 2), 0) // qdim
    c2 = jax.lax.broadcasted_iota(jnp.int32, (4 * qdim, 2), 1)
    w2 = (jnp.where(g2 == 2 * c2, 1.0, 0.0)
          - jnp.where(g2 == 2 * c2 + 1, 1.0, 0.0)).astype(jnp.float32)
    lr = jnp.dot(L, w2, preferred_element_type=jnp.float32)
    p = jnp.exp(jnp.minimum(lr, 0.0))
    x = p[:, 0:1] - conf[...]
    se_row = x * x + p[:, 1:2] * p[:, 1:2]

    sqs = [edh, edt, mh[...], mt[...], jnp.exp(rth), jnp.exp(rtt),
           sc_h, sc_t]
    Y = jnp.concatenate([a * a for a in sqs], axis=1)
    g8 = jax.lax.broadcasted_iota(jnp.int32, (8 * DIM, 8), 0) // DIM
    c8 = jax.lax.broadcasted_iota(jnp.int32, (8 * DIM, 8), 1)
    w8 = (g8 == c8).astype(jnp.float32)
    S = jnp.dot(Y, w8, preferred_element_type=jnp.float32)
    norms = jnp.sqrt(S)
    wreg = jnp.where(
        jax.lax.broadcasted_iota(jnp.int32, norms.shape, 1) < 2,
        REG_DELTA, REG_MIN)
    acc_ref[0] += jnp.sum(se_row) + jnp.sum(norms * wreg)

    @pl.when(i == nb - 1)
    def _():
        out_ref[...] = jnp.full((1, 1), acc_ref[0], jnp.float32)


def _make_tc_loss(nrows, split):
    nb = nrows // _BB
    off = split * nb
    k = _REL_ON_SC
    row_spec = pl.BlockSpec((_BB, DIM), lambda i: (i, 0))
    rel_spec = pl.BlockSpec((_NRELP, (4 - k) * DIM), lambda i: (0, 0))
    col_spec = pl.BlockSpec((_BB, 1), lambda i: (i + off, 0))
    in_specs = [row_spec] * (8 + k)
    if k < 4:
        in_specs += [rel_spec] * 2 + [col_spec]
    in_specs += [col_spec]
    return pl.pallas_call(
        functools.partial(_tc_loss_body, nb),
        grid=(nb,),
        in_specs=in_specs,
        out_specs=pl.BlockSpec((1, 1), lambda i: (0, 0)),
        out_shape=jax.ShapeDtypeStruct((1, 1), jnp.float32),
        scratch_shapes=[pltpu.SMEM((1,), jnp.float32)],
    )


def kernel(ids, negative_samples, confidence, min_embedding, delta_embedding,
           rel_trans_for_head, rel_scale_for_head, rel_trans_for_tail,
           rel_scale_for_tail):
    ids = ids.astype(jnp.int32)
    neg = negative_samples.astype(jnp.int32)
    h = ids[:, 0]
    t = ids[:, 2]
    nh = neg[:, 0]
    nt = neg[:, 2]
    r = ids[:, 1]
    n = B // _NSPLIT
    k = _REL_ON_SC
    conf2d = confidence.reshape(B, 1)
    rel_tables = [rel_trans_for_head, rel_scale_for_head,
                  rel_trans_for_tail, rel_scale_for_tail]
    if k < 4:
        r2d = r.reshape(B, 1)
        rel_cat = jnp.concatenate(rel_tables[k:], axis=1)
        rel_cat = jnp.pad(rel_cat, ((0, _NRELP - N_REL), (0, 0)))
        rel_hi = rel_cat.astype(jnp.bfloat16)
        rel_lo = (rel_cat - rel_hi.astype(jnp.float32)).astype(jnp.bfloat16)
    gathered_chunks = []
    for s in range(_NSPLIT):
        sc_args = ([min_embedding, delta_embedding] + rel_tables[:k]
                   + [h, t, nh, nt] + ([r] if k else []))
        gathered_chunks.append(_make_sc_gather(n, s)(*sc_args))
    partials = []
    for s in range(_NSPLIT):
        tc_args = list(gathered_chunks[s])
        if k < 4:
            tc_args += [rel_hi, rel_lo, r2d]
        tc_args.append(conf2d)
        partials.append(_make_tc_loss(n, s)(*tc_args))
    total = partials[0]
    for p in partials[1:]:
        total = total + p
    return (total * (1.0 / B)).reshape(())
